# baseline (speedup 1.0000x reference)
import functools

import jax
import jax.numpy as jnp
import numpy as np
from jax import lax
from jax.experimental import pallas as pl
from jax.experimental.pallas import tpu as pltpu
from jax.experimental.pallas import tpu_sc as plsc

_H = 128

_COL_OF_MEM = np.array(
    [32 * (m // 32) + (m % 32) // 2 + (16 if m % 2 else 0) for m in range(_H)],
    dtype=np.int32,
)


def _pre_body(feat_ref, w1a_ref, w1b_ref, b1m_ref, a_ref, b_ref):
    f = feat_ref[...]
    a_ref[...] = (
        jnp.dot(f, w1a_ref[...], preferred_element_type=jnp.float32)
        + b1m_ref[...][None, :]
    )
    b_ref[...] = jnp.dot(f, w1b_ref[...], preferred_element_type=jnp.float32)


def _post_body(acc_ref, cnt_ref, w2mp_ref, b2m_ref, w1n_ref, b1n_ref,
               w2n_ref, b2n_ref, out_ref):
    s = acc_ref[...].astype(jnp.float32)
    cnt = cnt_ref[...]
    mean = s / jnp.clip(cnt, 1.0)
    scalar = (
        jnp.dot(mean, w2mp_ref[...], preferred_element_type=jnp.float32)
        + b2m_ref[...][None, :]
    )
    scalar = jnp.where(cnt > 0.0, scalar, 0.0)
    t = (
        jnp.dot(scalar, w1n_ref[...], preferred_element_type=jnp.float32)
        + b1n_ref[...][None, :]
    )
    h2 = t * (1.0 / (1.0 + jnp.exp(-t)))
    out_ref[...] = (
        jnp.dot(h2, w2n_ref[...], preferred_element_type=jnp.float32)
        + b2n_ref[...][None, :]
    )


def _rne_bf16_hi(u):
    return lax.shift_right_logical(
        u + 0x7FFF + jnp.bitwise_and(lax.shift_right_logical(u, 16), 1), 16
    )


_HI_MASK = np.int32(-65536)


def _bf_lo(w):
    return plsc.bitcast(lax.shift_left(w, 16), jnp.float32)


def _bf_hi(w):
    return plsc.bitcast(jnp.bitwise_and(w, _HI_MASK), jnp.float32)


@functools.lru_cache(maxsize=None)
def _sc_edge_fn(EP, N, NN):
    NT = 16
    EPT = EP // NT
    C = 32
    NCH = EPT // C
    NP = -(-N // (16 * _H)) * (16 * _H)
    RPT = NP // 16
    NZ = RPT // C
    CR = NP // _H
    NZH = NN // 2

    mesh = plsc.VectorSubcoreMesh(core_axis_name="c", subcore_axis_name="s",
                                  num_cores=1)

    @functools.partial(
        pl.kernel,
        out_type=(
            jax.ShapeDtypeStruct((NP, _H // 2), jnp.int32),
            jax.ShapeDtypeStruct((CR, _H), jnp.float32),
        ),
        mesh=mesh,
        compiler_params=pltpu.CompilerParams(needs_layout_passes=False),
        scratch_types=[
            pltpu.VMEM((C,), jnp.int32),
            pltpu.VMEM((C,), jnp.int32),
            pltpu.VMEM((C, _H), jnp.float32),
            pltpu.VMEM((C, _H), jnp.float32),
            pltpu.VMEM((C,), jnp.float32),
            pltpu.VMEM((NN,), jnp.int32),
            pltpu.VMEM((NZH,), jnp.int32),
            pltpu.VMEM((_H,), jnp.float32),
            pltpu.VMEM((C, _H // 2), jnp.int32),
            pltpu.VMEM((CR, _H), jnp.float32),
            pltpu.VMEM((CR,), jnp.int32),
            pltpu.VMEM_SHARED((NP, _H), jnp.float32),
            pltpu.VMEM_SHARED((CR, _H), jnp.float32),
            pltpu.SemaphoreType.DMA,
            pltpu.SemaphoreType.DMA,
        ],
    )
    def sc_fn(a_hbm, b_hbm, xy_hbm, zp_hbm, row_hbm, col_hbm, wd_hbm,
              out_hbm, cnt_hbm,
              ridx_v, cidx_v, a_v, b_v, dist_v, xy_v, zp_v, wd_v,
              pk_v, cnt_v, rowi_v, acc_sh, cnt_sh, sem_a, sem_b):
        sid = lax.axis_index("s")
        base = sid * EPT

        pltpu.sync_copy(wd_hbm, wd_v)
        pltpu.sync_copy(xy_hbm, xy_v)
        pltpu.sync_copy(zp_hbm, zp_v)

        zeros16 = jnp.zeros((16,), jnp.float32)
        ones16 = jnp.ones((16,), jnp.float32)
        one_i = jnp.full((16,), 1, jnp.int32)
        lane = lax.iota(jnp.int32, 16)

        def zrow(r, carry):
            for f in range(_H // 16):
                a_v[r, pl.ds(f * 16, 16)] = zeros16
            return carry

        lax.fori_loop(0, C, zrow, 0)

        def zcnt(r, carry):
            for f in range(_H // 16):
                cnt_v[r, pl.ds(f * 16, 16)] = zeros16
            return carry

        lax.fori_loop(0, CR, zcnt, 0)
        row0 = sid * RPT
        for j in range(NZ):
            pltpu.sync_copy(a_v, acc_sh.at[pl.ds(row0 + j * C, C)])

        @pl.when(sid == 0)
        def _():
            pltpu.sync_copy(a_v, cnt_sh.at[pl.ds(0, C)])
            pltpu.sync_copy(a_v, cnt_sh.at[pl.ds(C, C)])
            pltpu.sync_copy(a_v.at[pl.ds(0, CR - 2 * C)],
                            cnt_sh.at[pl.ds(2 * C, CR - 2 * C)])

        def rowi_init(g, carry):
            rowi_v[pl.ds(g * 16, 16)] = lane + g * 16
            return carry

        lax.fori_loop(0, CR // 16, rowi_init, 0)

        plsc.subcore_barrier()

        wd_regs = [wd_v[pl.ds(f * 16, 16)] for f in range(_H // 16)]

        def chunk_body(i, carry):
            off = base + i * C
            pltpu.sync_copy(row_hbm.at[pl.ds(off, C)], ridx_v)
            pltpu.sync_copy(col_hbm.at[pl.ds(off, C)], cidx_v)
            cp_a = pltpu.async_copy(a_hbm.at[ridx_v], a_v, sem_a)
            cp_b = pltpu.async_copy(b_hbm.at[cidx_v], b_v, sem_b)

            for g in range(C // 16):
                ri = ridx_v[pl.ds(g * 16, 16)]
                ci = cidx_v[pl.ds(g * 16, 16)]
                plsc.addupdate_scatter(
                    cnt_v,
                    [lax.shift_right_logical(ri, 7), jnp.bitwise_and(ri, 127)],
                    ones16,
                )
                wr = plsc.load_gather(xy_v, [ri])
                wc = plsc.load_gather(xy_v, [ci])
                dx = _bf_lo(wr) - _bf_lo(wc)
                dy = _bf_hi(wr) - _bf_hi(wc)
                zr_w = plsc.load_gather(zp_v, [lax.shift_right_logical(ri, 1)])
                zc_w = plsc.load_gather(zp_v, [lax.shift_right_logical(ci, 1)])
                zr = jnp.where(jnp.bitwise_and(ri, 1) == 1,
                               _bf_hi(zr_w), _bf_lo(zr_w))
                zc = jnp.where(jnp.bitwise_and(ci, 1) == 1,
                               _bf_hi(zc_w), _bf_lo(zc_w))
                dz = zr - zc
                d2 = dx * dx + dy * dy + dz * dz
                x = jnp.maximum(d2, 1e-16)
                ib = plsc.bitcast(x, jnp.int32)
                ib = 0x5F3759DF - lax.shift_right_logical(ib, 1)
                y = plsc.bitcast(ib, jnp.float32)
                for _ in range(4):
                    y = y * (1.5 - 0.5 * x * y * y)
                dist_v[pl.ds(g * 16, 16)] = jnp.maximum(x * y, 1e-8)

            cp_a.wait()
            cp_b.wait()

            def edge_body(e, carry2):
                esplat = jnp.full((16,), 0, jnp.int32) + e
                d = plsc.load_gather(dist_v, [esplat])
                for f in range(_H // 16):
                    v = (a_v[e, pl.ds(f * 16, 16)]
                         + b_v[e, pl.ds(f * 16, 16)]
                         + d * wd_regs[f])
                    a_v[e, pl.ds(f * 16, 16)] = v / (1.0 + jnp.exp(-v))
                return carry2

            lax.fori_loop(0, C, edge_body, 0)

            pltpu.sync_copy(a_v, acc_sh.at[ridx_v], add=True)
            return carry

        lax.fori_loop(0, NCH, chunk_body, 0)

        pltpu.sync_copy(cnt_v, cnt_sh.at[rowi_v], add=True)
        plsc.subcore_barrier()

        for j in range(NZ):
            r = row0 + j * C
            pltpu.sync_copy(acc_sh.at[pl.ds(r, C)], b_v)

            def pkrow(q, carry):
                for f in range(_H // 32):
                    x0 = b_v[q, pl.ds(f * 32, 16)]
                    x1 = b_v[q, pl.ds(f * 32 + 16, 16)]
                    r0 = _rne_bf16_hi(plsc.bitcast(x0, jnp.int32))
                    r1 = _rne_bf16_hi(plsc.bitcast(x1, jnp.int32))
                    w = jnp.bitwise_or(r0, lax.shift_left(r1, 16))
                    pk_v[q, pl.ds(f * 16, 16)] = w
                return carry

            lax.fori_loop(0, C, pkrow, 0)
            pltpu.sync_copy(pk_v, out_hbm.at[pl.ds(r, C)])

        @pl.when(sid == 0)
        def _():
            pltpu.sync_copy(cnt_sh, cnt_hbm)

    return sc_fn


def kernel(node_feat, node_pos, batch, edge_index, W1m, b1m, W2m, b2m,
           W1n, b1n, W2n, b2n):
    del batch
    N, H = node_feat.shape
    E = edge_index.shape[1]
    NN = N + 16
    EP = -(-E // 512) * 512

    w1a = W1m[:H]
    w1b = W1m[H:2 * H]
    wd = W1m[2 * H]

    featp = jnp.concatenate(
        [node_feat, jnp.zeros((NN - N, H), jnp.float32)], axis=0
    )
    a, b = pl.pallas_call(
        _pre_body,
        out_shape=(
            jax.ShapeDtypeStruct((NN, _H), jnp.float32),
            jax.ShapeDtypeStruct((NN, _H), jnp.float32),
        ),
    )(featp, w1a, w1b, b1m)

    posp = jnp.concatenate(
        [node_pos.astype(jnp.float32), jnp.zeros((NN - N, 3), jnp.float32)]
    )
    pb = lax.bitcast_convert_type(
        posp.astype(jnp.bfloat16), jnp.uint16
    ).astype(jnp.int32)
    xy = pb[:, 0] | (pb[:, 1] << 16)
    zp = pb[0::2, 2] | (pb[1::2, 2] << 16)

    ei = edge_index.astype(jnp.int32)
    rowp = jnp.concatenate([ei[0], jnp.full((EP - E,), N, jnp.int32)])
    colp = jnp.concatenate([ei[1], jnp.zeros((EP - E,), jnp.int32)])

    accpk, cntarr = _sc_edge_fn(EP, N, NN)(a, b, xy, zp, rowp, colp, wd)

    acc = lax.bitcast_convert_type(accpk, jnp.bfloat16).reshape(
        accpk.shape[0], -1
    )
    acc = acc[:N]
    cnt = cntarr.reshape(-1, 1)[:N]
    w2mp = W2m[jnp.asarray(_COL_OF_MEM)]

    out = pl.pallas_call(
        _post_body,
        out_shape=jax.ShapeDtypeStruct((N, _H), jnp.float32),
    )(acc, cnt, w2mp, b2m, W1n, b1n, W2n, b2n)
    return out


# speedup vs baseline: 1.1922x; 1.1922x over previous
import functools

import jax
import jax.numpy as jnp
import numpy as np
from jax import lax
from jax.experimental import pallas as pl
from jax.experimental.pallas import tpu as pltpu
from jax.experimental.pallas import tpu_sc as plsc

_H = 128

_COL_OF_MEM = np.array(
    [32 * (m // 32) + (m % 32) // 2 + (16 if m % 2 else 0) for m in range(_H)],
    dtype=np.int32,
)


def _pre_body(feat_ref, w1a_ref, w1b_ref, b1m_ref, a_ref, b_ref):
    f = feat_ref[...]
    a_ref[...] = (
        jnp.dot(f, w1a_ref[...], preferred_element_type=jnp.float32)
        + b1m_ref[...][None, :]
    )
    b_ref[...] = jnp.dot(f, w1b_ref[...], preferred_element_type=jnp.float32)


def _post_body(acc_ref, cnt_ref, w2mp_ref, b2m_ref, w1n_ref, b1n_ref,
               w2n_ref, b2n_ref, out_ref):
    s = acc_ref[...].astype(jnp.float32)
    cnt = cnt_ref[...]
    mean = s / jnp.clip(cnt, 1.0)
    scalar = (
        jnp.dot(mean, w2mp_ref[...], preferred_element_type=jnp.float32)
        + b2m_ref[...][None, :]
    )
    scalar = jnp.where(cnt > 0.0, scalar, 0.0)
    t = (
        jnp.dot(scalar, w1n_ref[...], preferred_element_type=jnp.float32)
        + b1n_ref[...][None, :]
    )
    h2 = t * (1.0 / (1.0 + jnp.exp(-t)))
    out_ref[...] = (
        jnp.dot(h2, w2n_ref[...], preferred_element_type=jnp.float32)
        + b2n_ref[...][None, :]
    )


def _rne_bf16_hi(u):
    return lax.shift_right_logical(
        u + 0x7FFF + jnp.bitwise_and(lax.shift_right_logical(u, 16), 1), 16
    )


_HI_MASK = np.int32(-65536)


def _bf_lo(w):
    return plsc.bitcast(lax.shift_left(w, 16), jnp.float32)


def _bf_hi(w):
    return plsc.bitcast(jnp.bitwise_and(w, _HI_MASK), jnp.float32)


@functools.lru_cache(maxsize=None)
def _sc_edge_fn(EP, N, NN):
    NT = 16
    EPT = EP // NT
    C = 32
    BC = 8
    NCH = EPT // C
    NBLK = NCH // BC
    NP = -(-N // (16 * _H)) * (16 * _H)
    RPT = NP // 16
    NZ = RPT // C
    CR = 48
    NZH = NN // 2

    mesh = plsc.VectorSubcoreMesh(core_axis_name="c", subcore_axis_name="s",
                                  num_cores=1)

    @functools.partial(
        pl.kernel,
        out_type=(
            jax.ShapeDtypeStruct((NP, _H // 2), jnp.int32),
            jax.ShapeDtypeStruct((CR, _H), jnp.int32),
        ),
        mesh=mesh,
        compiler_params=pltpu.CompilerParams(needs_layout_passes=False),
        scratch_types=[
            pltpu.VMEM((BC, C), jnp.int32),
            pltpu.VMEM((BC, C), jnp.int32),
            pltpu.VMEM((BC, C), jnp.int32),
            pltpu.VMEM((BC, C), jnp.int32),
            pltpu.VMEM((C, _H), jnp.float32),
            pltpu.VMEM((C, _H), jnp.float32),
            pltpu.VMEM((C, _H), jnp.float32),
            pltpu.VMEM((C, _H), jnp.float32),
            pltpu.VMEM((C,), jnp.float32),
            pltpu.VMEM((NN,), jnp.int32),
            pltpu.VMEM((NZH,), jnp.int32),
            pltpu.VMEM((_H,), jnp.float32),
            pltpu.VMEM((C, _H // 2), jnp.int32),
            pltpu.VMEM((CR, _H), jnp.int32),
            pltpu.VMEM((CR,), jnp.int32),
            pltpu.VMEM((C,), jnp.int32),
            pltpu.VMEM_SHARED((NP, _H), jnp.float32),
            pltpu.VMEM_SHARED((CR, _H), jnp.int32),
            pltpu.SemaphoreType.DMA,
            pltpu.SemaphoreType.DMA,
            pltpu.SemaphoreType.DMA,
            pltpu.SemaphoreType.DMA,
            pltpu.SemaphoreType.DMA,
            pltpu.SemaphoreType.DMA,
            pltpu.SemaphoreType.DMA,
            pltpu.SemaphoreType.DMA,
            pltpu.SemaphoreType.DMA,
            pltpu.SemaphoreType.DMA,
        ],
    )
    def sc_fn(a_hbm, b_hbm, xy_hbm, zp_hbm, row_hbm, col_hbm, wd_hbm,
              out_hbm, cnt_hbm,
              ridx0_v, cidx0_v, ridx1_v, cidx1_v, a0_v, a1_v, b0_v, b1_v,
              dist_v, xy_v, zp_v, wd_v, pk_v, cnt_v, rowi_v, prime_v,
              acc_sh, cnt_sh,
              sem_a0, sem_a1, sem_b0, sem_b1, sem_s0, sem_s1,
              sem_ir0, sem_ic0, sem_ir1, sem_ic1):
        sid = lax.axis_index("s")
        a_bufs = (a0_v, a1_v)
        b_bufs = (b0_v, b1_v)
        ridxs = (ridx0_v, ridx1_v)
        cidxs = (cidx0_v, cidx1_v)
        sems_a = (sem_a0, sem_a1)
        sems_b = (sem_b0, sem_b1)
        sems_s = (sem_s0, sem_s1)
        sems_ir = (sem_ir0, sem_ir1)
        sems_ic = (sem_ic0, sem_ic1)

        pltpu.sync_copy(wd_hbm, wd_v)
        pltpu.sync_copy(xy_hbm, xy_v)
        pltpu.sync_copy(zp_hbm, zp_v)

        zeros16 = jnp.zeros((16,), jnp.float32)
        zeros16i = jnp.zeros((16,), jnp.int32)
        lane = lax.iota(jnp.int32, 16)

        def zrow(r, carry):
            for f in range(_H // 16):
                a0_v[r, pl.ds(f * 16, 16)] = zeros16
                a1_v[r, pl.ds(f * 16, 16)] = zeros16
            return carry

        lax.fori_loop(0, C, zrow, 0)

        def zcnt(r, carry):
            for f in range(_H // 16):
                cnt_v[r, pl.ds(f * 16, 16)] = zeros16i
            return carry

        lax.fori_loop(0, CR, zcnt, 0)
        row0 = sid * RPT
        for j in range(NZ):
            pltpu.sync_copy(a0_v, acc_sh.at[pl.ds(row0 + j * C, C)])

        @pl.when(sid == 0)
        def _():
            pltpu.sync_copy(cnt_v, cnt_sh)

        def rowi_init(g, carry):
            rowi_v[pl.ds(g * 16, 16)] = lane + g * 16
            return carry

        lax.fori_loop(0, CR // 16, rowi_init, 0)

        def prime_init(g, carry):
            prime_v[pl.ds(g * 16, 16)] = lane + g * 16
            return carry

        lax.fori_loop(0, C // 16, prime_init, 0)

        plsc.subcore_barrier()

        wd_regs = [wd_v[pl.ds(f * 16, 16)] for f in range(_H // 16)]

        pltpu.async_copy(a0_v, acc_sh.at[prime_v], sem_s0, add=True)
        pltpu.async_copy(a1_v, acc_sh.at[prime_v], sem_s1, add=True)

        def chunk_compute(k, j, p):
            ridx_v = ridxs[k]
            cidx_v = cidxs[k]
            av = a_bufs[p]
            bv = b_bufs[p]
            for g in range(C // 16):
                ri = ridx_v[j, pl.ds(g * 16, 16)]
                ci = cidx_v[j, pl.ds(g * 16, 16)]
                inc = jnp.where(jnp.bitwise_and(ri, 1) == 1, 65536, 1)
                plsc.addupdate_scatter(
                    cnt_v,
                    [lax.shift_right_logical(ri, 8),
                     jnp.bitwise_and(lax.shift_right_logical(ri, 1), 127)],
                    inc,
                )
                wr = plsc.load_gather(xy_v, [ri])
                wc = plsc.load_gather(xy_v, [ci])
                dx = _bf_lo(wr) - _bf_lo(wc)
                dy = _bf_hi(wr) - _bf_hi(wc)
                zr_w = plsc.load_gather(zp_v, [lax.shift_right_logical(ri, 1)])
                zc_w = plsc.load_gather(zp_v, [lax.shift_right_logical(ci, 1)])
                zr = jnp.where(jnp.bitwise_and(ri, 1) == 1,
                               _bf_hi(zr_w), _bf_lo(zr_w))
                zc = jnp.where(jnp.bitwise_and(ci, 1) == 1,
                               _bf_hi(zc_w), _bf_lo(zc_w))
                dz = zr - zc
                d2 = dx * dx + dy * dy + dz * dz
                x = jnp.maximum(d2, 1e-16)
                ib = plsc.bitcast(x, jnp.int32)
                ib = 0x5F3759DF - lax.shift_right_logical(ib, 1)
                y = plsc.bitcast(ib, jnp.float32)
                for _ in range(4):
                    y = y * (1.5 - 0.5 * x * y * y)
                dist_v[pl.ds(g * 16, 16)] = jnp.maximum(x * y, 1e-8)

            def edge_body(e, carry2):
                esplat = jnp.full((16,), 0, jnp.int32) + e
                d = plsc.load_gather(dist_v, [esplat])
                for f in range(_H // 16):
                    v = (av[e, pl.ds(f * 16, 16)]
                         + bv[e, pl.ds(f * 16, 16)]
                         + d * wd_regs[f])
                    av[e, pl.ds(f * 16, 16)] = v / (1.0 + jnp.exp(-v))
                return carry2

            lax.fori_loop(0, C, edge_body, 0)

        def idx_rows(bn):
            return sid * NCH + bn * BC

        def start_idx_fetch(bn, k):
            pltpu.async_copy(row_hbm.at[pl.ds(idx_rows(bn), BC)], ridxs[k],
                             sems_ir[k])
            pltpu.async_copy(col_hbm.at[pl.ds(idx_rows(bn), BC)], cidxs[k],
                             sems_ic[k])

        def run_block(bn, k):
            ridx_v = ridxs[k]
            cidx_v = cidxs[k]
            pltpu.make_async_copy(row_hbm.at[pl.ds(idx_rows(bn), BC)],
                                  ridx_v, sems_ir[k]).wait()
            pltpu.make_async_copy(col_hbm.at[pl.ds(idx_rows(bn), BC)],
                                  cidx_v, sems_ic[k]).wait()
            start_idx_fetch(bn + 1, 1 - k)

            pltpu.make_async_copy(a_hbm.at[pl.ds(0, C)], a_bufs[0],
                                  sems_s[0]).wait()
            pltpu.async_copy(a_hbm.at[ridx_v.at[0]], a_bufs[0], sems_a[0])
            pltpu.async_copy(b_hbm.at[cidx_v.at[0]], b_bufs[0], sems_b[0])
            for j in range(BC):
                p = j % 2
                q = 1 - p
                if j < BC - 1:
                    pltpu.make_async_copy(a_hbm.at[pl.ds(0, C)], a_bufs[q],
                                          sems_s[q]).wait()
                    pltpu.async_copy(a_hbm.at[ridx_v.at[j + 1]], a_bufs[q],
                                     sems_a[q])
                    pltpu.async_copy(b_hbm.at[cidx_v.at[j + 1]], b_bufs[q],
                                     sems_b[q])
                pltpu.make_async_copy(a_hbm.at[ridx_v.at[j]], a_bufs[p],
                                      sems_a[p]).wait()
                pltpu.make_async_copy(b_hbm.at[cidx_v.at[j]], b_bufs[p],
                                      sems_b[p]).wait()
                chunk_compute(k, j, p)
                pltpu.async_copy(a_bufs[p], acc_sh.at[ridx_v.at[j]],
                                 sems_s[p], add=True)

        start_idx_fetch(0, 0)

        def pair_body(b2, carry):
            run_block(2 * b2, 0)
            run_block(2 * b2 + 1, 1)
            return carry

        lax.fori_loop(0, NBLK // 2, pair_body, 0)

        pltpu.make_async_copy(a_hbm.at[pl.ds(0, C)], a_bufs[0],
                              sems_s[0]).wait()
        pltpu.make_async_copy(a_hbm.at[pl.ds(0, C)], a_bufs[1],
                              sems_s[1]).wait()
        pltpu.make_async_copy(row_hbm.at[pl.ds(0, BC)], ridxs[0],
                              sems_ir[0]).wait()
        pltpu.make_async_copy(col_hbm.at[pl.ds(0, BC)], cidxs[0],
                              sems_ic[0]).wait()

        pltpu.sync_copy(cnt_v, cnt_sh.at[rowi_v], add=True)
        plsc.subcore_barrier()

        for j in range(NZ):
            r = row0 + j * C
            pltpu.sync_copy(acc_sh.at[pl.ds(r, C)], b0_v)

            def pkrow(q, carry):
                for f in range(_H // 32):
                    x0 = b0_v[q, pl.ds(f * 32, 16)]
                    x1 = b0_v[q, pl.ds(f * 32 + 16, 16)]
                    r0 = _rne_bf16_hi(plsc.bitcast(x0, jnp.int32))
                    r1 = _rne_bf16_hi(plsc.bitcast(x1, jnp.int32))
                    w = jnp.bitwise_or(r0, lax.shift_left(r1, 16))
                    pk_v[q, pl.ds(f * 16, 16)] = w
                return carry

            lax.fori_loop(0, C, pkrow, 0)
            pltpu.sync_copy(pk_v, out_hbm.at[pl.ds(r, C)])

        @pl.when(sid == 0)
        def _():
            pltpu.sync_copy(cnt_sh, cnt_hbm)

    return sc_fn


def kernel(node_feat, node_pos, batch, edge_index, W1m, b1m, W2m, b2m,
           W1n, b1n, W2n, b2n):
    del batch
    N, H = node_feat.shape
    E = edge_index.shape[1]
    NN = N + 16
    EP = -(-E // 8192) * 8192

    w1a = W1m[:H]
    w1b = W1m[H:2 * H]
    wd = W1m[2 * H]

    featp = jnp.concatenate(
        [node_feat, jnp.zeros((NN - N, H), jnp.float32)], axis=0
    )
    a, b = pl.pallas_call(
        _pre_body,
        out_shape=(
            jax.ShapeDtypeStruct((NN, _H), jnp.float32),
            jax.ShapeDtypeStruct((NN, _H), jnp.float32),
        ),
    )(featp, w1a, w1b, b1m)

    posp = jnp.concatenate(
        [node_pos.astype(jnp.float32), jnp.zeros((NN - N, 3), jnp.float32)]
    )
    pb = lax.bitcast_convert_type(
        posp.astype(jnp.bfloat16), jnp.uint16
    ).astype(jnp.int32)
    xy = pb[:, 0] | (pb[:, 1] << 16)
    zp = pb[0::2, 2] | (pb[1::2, 2] << 16)

    ei = edge_index.astype(jnp.int32)
    rowp = jnp.concatenate([ei[0], jnp.full((EP - E,), N, jnp.int32)])
    colp = jnp.concatenate([ei[1], jnp.zeros((EP - E,), jnp.int32)])
    rowp2 = jnp.concatenate(
        [rowp.reshape(-1, 32), jnp.full((8, 32), N, jnp.int32)]
    )
    colp2 = jnp.concatenate(
        [colp.reshape(-1, 32), jnp.zeros((8, 32), jnp.int32)]
    )

    accpk, cntarr = _sc_edge_fn(EP, N, NN)(a, b, xy, zp, rowp2, colp2, wd)

    acc = lax.bitcast_convert_type(accpk, jnp.bfloat16).reshape(
        accpk.shape[0], -1
    )
    acc = acc[:N]
    lo = jnp.bitwise_and(cntarr, 65535)
    hi = lax.shift_right_logical(cntarr, 16)
    cnt = jnp.stack([lo, hi], axis=-1).reshape(-1)[:N]
    cnt = cnt.astype(jnp.float32)[:, None]
    w2mp = W2m[jnp.asarray(_COL_OF_MEM)]

    out = pl.pallas_call(
        _post_body,
        out_shape=jax.ShapeDtypeStruct((N, _H), jnp.float32),
    )(acc, cnt, w2mp, b2m, W1n, b1n, W2n, b2n)
    return out
